# baseline (speedup 1.0000x reference)
import numpy as np

import jax
import jax.numpy as jnp
from jax import lax
from jax.experimental import pallas as pl
from jax.experimental.pallas import tpu as pltpu
from jax.experimental.pallas import tpu_sc as plsc

N_NODES = 10000
N_EDGES = 320000
D_FEAT = 128
DH = D_FEAT // 2
NT = 16
C = 128
NI = 157
EP = NT * NI * C
NG = (NI - 5) // 4
RPT = 624

_SIGMA = np.array(list(range(0, 32, 2)) + list(range(1, 32, 2)) +
                  list(range(32, 64, 2)) + list(range(33, 64, 2)))
_PBLK = np.empty(64, np.int32)
_PBLK[_SIGMA] = np.arange(64, dtype=np.int32)
_PERM = np.concatenate([_PBLK, _PBLK + 64])

_SIG32 = np.array(list(range(0, 32, 2)) + list(range(1, 32, 2)))
_P32 = np.empty(32, np.int32)
_P32[_SIG32] = np.arange(32, dtype=np.int32)

_GATHER_DNUMS = lax.GatherDimensionNumbers(
    offset_dims=(), collapsed_slice_dims=(0,), start_index_map=(0,))


def _lane_bcast(vec, lane):
    idx = jnp.full((16, 1), lane, jnp.int32)
    return lax.gather(vec, idx, _GATHER_DNUMS, slice_sizes=(1,),
                      mode=lax.GatherScatterMode.PROMISE_IN_BOUNDS)


def _body(x2, srcA, srcB, dst3, w3, out, acc, srcb, dstb, wb,
          dc0, dc1, dc2, dc3, g0, g1, g2, g3, f0, f1, f2, f3,
          gs0, gs1, gs2, gs3, ss0, ss1, ss2, ss3):
    c = lax.axis_index("c")
    s = lax.axis_index("s")
    r0 = s * RPT
    dstc = (dc0, dc1, dc2, dc3)
    gbuf = (g0, g1, g2, g3)
    fbuf = (f0, f1, f2, f3)
    gsem = (gs0, gs1, gs2, gs3)
    ssem = (ss0, ss1, ss2, ss3)

    def drain_g(b):
        pltpu.make_async_copy(x2.at[pl.ds(0, C)], gbuf[b], gsem[b]).wait()

    def drain_s(b):
        pltpu.make_async_copy(fbuf[b], acc.at[pl.ds(0, C)], ssem[b]).wait()

    def gather(i, b):
        pltpu.async_copy(x2.at[srcb.at[i]], gbuf[b], gsem[b])

    def scatter(i, b):
        for j in range(C // 32):
            d32 = dstb[i, pl.ds(j * 32, 32)]
            da, db = plsc.unpack(d32, format=plsc.PackFormat.INTERLEAVED,
                                 preferred_element_type=jnp.int32)
            dstc[b][pl.ds(j * 32, 16)] = da
            dstc[b][pl.ds(j * 32 + 16, 16)] = db
        pltpu.async_copy(fbuf[b], acc.at[dstc[b]], ssem[b], add=True)

    def compute(i, b):
        gb, fb = gbuf[b], fbuf[b]
        for j in range(C // 32):
            w32 = wb[i, pl.ds(j * 32, 32)]
            wa, wo = plsc.unpack(w32, format=plsc.PackFormat.INTERLEAVED,
                                 preferred_element_type=jnp.float32)
            for m in range(32):
                r = j * 32 + m
                wk = _lane_bcast(wa if m % 2 == 0 else wo, m // 2)
                for h in range(DH // 32):
                    v = gb[r, pl.ds(h * 32, 32)]
                    a, bb = plsc.unpack(v, format=plsc.PackFormat.INTERLEAVED,
                                        preferred_element_type=jnp.float32)
                    fb[r, pl.ds(h * 32, 16)] = a * wk
                    fb[r, pl.ds(h * 32 + 16, 16)] = bb * wk

    def chunk_step(i, b, first, traced):
        drain_g(b)
        bn = (b + 2) % 4
        if traced:
            @pl.when(i + 2 < NI)
            def _g():
                gather(i + 2, bn)
        elif i + 2 < NI:
            gather(i + 2, bn)
        if traced:
            @pl.when(i >= 4)
            def _ds():
                drain_s(b)
        elif not first:
            drain_s(b)
        compute(i, b)
        scatter(i, b)

    @pl.when(c == 0)
    def _psA():
        pltpu.sync_copy(srcA.at[s], srcb)

    @pl.when(c == 1)
    def _psB():
        pltpu.sync_copy(srcB.at[s], srcb)

    pltpu.sync_copy(dst3.at[s], dstb)
    pltpu.sync_copy(w3.at[s], wb)

    gather(0, 0)
    gather(1, 1)

    zero = jnp.zeros((16,), jnp.float32)

    def zrow(r, carry):
        for q in range(DH // 16):
            f0[r, pl.ds(q * 16, 16)] = zero
        return carry

    lax.fori_loop(0, C, zrow, None)
    for k in range(RPT // C):
        pltpu.sync_copy(f0.at[:], acc.at[pl.ds(r0 + k * C, C)])
    tail = RPT % C
    pltpu.sync_copy(f0.at[pl.ds(0, tail)],
                    acc.at[pl.ds(r0 + (RPT // C) * C, tail)])
    rem = N_NODES - NT * RPT

    @pl.when(s == NT - 1)
    def _zero_rem():
        pltpu.sync_copy(f0.at[pl.ds(0, rem)],
                        acc.at[pl.ds(NT * RPT, rem)])

    plsc.subcore_barrier()

    chunk_step(0, 0, True, False)
    chunk_step(1, 1, True, False)

    def group(g, carry):
        i0 = 4 * g + 2
        for u in range(4):
            i = i0 + u
            chunk_step(i, (2 + u) % 4, False, True)
        return carry

    lax.fori_loop(0, NG, group, None)
    chunk_step(NI - 3, (NI - 3) % 4, False, False)
    chunk_step(NI - 2, (NI - 2) % 4, False, False)
    chunk_step(NI - 1, (NI - 1) % 4, False, False)
    for k in range(4):
        drain_s((NI - 4 + k) % 4)

    plsc.subcore_barrier()

    pltpu.sync_copy(acc.at[pl.ds(r0, RPT)],
                    out.at[pl.ds(r0, RPT), pl.ds(c * DH, DH)])

    @pl.when(s == NT - 1)
    def _write_rem():
        pltpu.sync_copy(acc.at[pl.ds(NT * RPT, rem)],
                        out.at[pl.ds(NT * RPT, rem), pl.ds(c * DH, DH)])


_sc_spmm = pl.kernel(
    _body,
    out_type=jax.ShapeDtypeStruct((N_NODES, D_FEAT), jnp.float32),
    mesh=plsc.VectorSubcoreMesh(core_axis_name="c", subcore_axis_name="s"),
    scratch_types=(
        [pltpu.VMEM_SHARED((N_NODES, DH), jnp.float32)] +
        [pltpu.VMEM((NI, C), jnp.int32)] +
        [pltpu.VMEM((NI, C), jnp.int16)] +
        [pltpu.VMEM((NI, C), jnp.bfloat16)] +
        [pltpu.VMEM((C,), jnp.int32)] * 4 +
        [pltpu.VMEM((C, DH), jnp.bfloat16)] * 4 +
        [pltpu.VMEM((C, DH), jnp.float32)] * 4 +
        [pltpu.SemaphoreType.DMA] * 8
    ),
    compiler_params=pltpu.CompilerParams(use_tc_tiling_on_sc=False,
                                         needs_layout_passes=False),
)


@jax.jit
def kernel(x, edge_index, edge_weight):
    pad = EP - N_EDGES
    s0 = jnp.pad(edge_index[0], (0, pad))
    d0 = jnp.pad(edge_index[1], (0, pad))
    w0 = jnp.pad(edge_weight, (0, pad))
    srcA = (s0 * 2).reshape(NT, NI, C)
    srcB = (s0 * 2 + 1).reshape(NT, NI, C)
    dst = (d0.reshape(-1, 32)[:, _P32].astype(jnp.int16)
           .reshape(NT, NI, C))
    w = w0.astype(jnp.bfloat16).reshape(NT, NI, C)
    xp = x[:, _PERM].astype(jnp.bfloat16)
    x2 = xp.reshape(2 * N_NODES, DH)
    return _sc_spmm(x2, srcA, srcB, dst, w)
